# baseline (speedup 1.0000x reference)
import functools

import jax
import jax.numpy as jnp
from jax.experimental import pallas as pl
from jax.experimental.pallas import tpu as pltpu

LRELU_SLOPE = 0.01
VMEM_LIMIT = 48 * 1024 * 1024


def _lrelu(y):
    return jnp.where(y > 0, y, LRELU_SLOPE * y)


def _conv3(x, w0, w1, w2, scale, shift, is_start, is_end, act):
    m = x.shape[0]
    xb = x if x.dtype == jnp.bfloat16 else x.astype(jnp.bfloat16)
    y1 = jnp.dot(xb, w1, preferred_element_type=jnp.float32)
    y0 = jnp.dot(xb, w0, preferred_element_type=jnp.float32)
    y2 = jnp.dot(xb, w2, preferred_element_type=jnp.float32)
    acc = (y1
           + jnp.where(is_start, 0.0, pltpu.roll(y0, 1, axis=0))
           + jnp.where(is_end, 0.0, pltpu.roll(y2, m - 1, axis=0)))
    y = acc * scale + shift
    return _lrelu(y) if act else y


def _trunk_kernel(x_ref, *refs, seq_len):
    wr = refs[:60]
    rc_o, cm_o = refs[60], refs[61]
    s50, s256, s512, s_sh, s128 = refs[62:]

    m = x_ref.shape[0]
    row = jax.lax.broadcasted_iota(jnp.int32, (m, 1), 0)
    is_start = (row % seq_len) == 0
    is_end = (row % seq_len) == (seq_len - 1)

    def layer(src_ref, dst_ref, li, act):
        w0, w1, w2, scale, shift = (r[...] for r in wr[5 * li:5 * li + 5])
        y = _conv3(src_ref[...], w0, w1, w2, scale, shift,
                   is_start, is_end, act)
        dst_ref[...] = y.astype(dst_ref.dtype)

    layer(x_ref, s50, 0, True)
    layer(s50, s256, 1, True)
    layer(s256, s512, 2, True)
    layer(s512, s_sh, 3, True)
    layer(s_sh, s512, 4, True)
    layer(s512, s256, 5, True)
    layer(s256, s50, 6, True)
    layer(s50, rc_o, 7, False)
    layer(s_sh, s512, 8, True)
    layer(s512, s256, 9, True)
    layer(s256, s128, 10, True)
    layer(s128, cm_o, 11, True)


def _trunk(x_2d, layer_params, seq_len):
    M, cin0 = x_2d.shape
    n_blocks = 2 if M % (2 * seq_len) == 0 else 1
    bm = M // n_blocks

    in_specs = [pl.BlockSpec((bm, cin0), lambda i: (i, 0))]
    args = [x_2d]
    for p in layer_params:
        for a in p:
            in_specs.append(pl.BlockSpec(a.shape, lambda i: (0, 0)))
            args.append(a)

    kern = functools.partial(_trunk_kernel, seq_len=seq_len)
    return pl.pallas_call(
        kern,
        out_shape=[jax.ShapeDtypeStruct((M, 2), jnp.float32),
                   jax.ShapeDtypeStruct((M, 50), jnp.bfloat16)],
        grid=(n_blocks,),
        in_specs=in_specs,
        out_specs=[pl.BlockSpec((bm, 2), lambda i: (i, 0)),
                   pl.BlockSpec((bm, 50), lambda i: (i, 0))],
        scratch_shapes=[pltpu.VMEM((bm, 50), jnp.bfloat16),
                        pltpu.VMEM((bm, 256), jnp.bfloat16),
                        pltpu.VMEM((bm, 512), jnp.bfloat16),
                        pltpu.VMEM((bm, 1024), jnp.bfloat16),
                        pltpu.VMEM((bm, 128), jnp.bfloat16)],
        compiler_params=pltpu.CompilerParams(
            dimension_semantics=("arbitrary",),
            vmem_limit_bytes=VMEM_LIMIT),
    )(*args)


def _lstm_cls_kernel(x_ref, wih1_ref, b1_ref, whh1_ref, w2cat_ref, b2_ref,
                     w1r_ref,
                     c0b_ref, c1w_ref, c1b_ref, c2w_ref, c2b_ref,
                     c3w_ref, c3b_ref,
                     logits_ref, rnn_ref,
                     gx_ref, h1_ref, c1_ref, h2_ref, c2_ref, acc_ref,
                     *, T, H, BP, K):
    s = pl.program_id(0)
    S = T // K

    @pl.when(s == 0)
    def _():
        h1_ref[...] = jnp.zeros_like(h1_ref)
        c1_ref[...] = jnp.zeros_like(c1_ref)
        h2_ref[...] = jnp.zeros_like(h2_ref)
        c2_ref[...] = jnp.zeros_like(c2_ref)
        acc_ref[...] = jnp.zeros_like(acc_ref)
        gx_ref[...] = jnp.dot(x_ref[...].astype(jnp.bfloat16), wih1_ref[...],
                              preferred_element_type=jnp.float32) + b1_ref[...]

    def cell(g, c):
        i_g = jax.nn.sigmoid(g[:, 0:H])
        f_g = jax.nn.sigmoid(g[:, H:2 * H])
        g_g = jnp.tanh(g[:, 2 * H:3 * H])
        o_g = jax.nn.sigmoid(g[:, 3 * H:4 * H])
        c_new = f_g * c + i_g * g_g
        return o_g * jnp.tanh(c_new), c_new

    h1, c1 = h1_ref[...], c1_ref[...]
    h2, c2 = h2_ref[...], c2_ref[...]
    hs = []
    for j in range(K):
        row = pl.multiple_of((s * K + j) * BP, BP)
        g1 = gx_ref[pl.ds(row, BP), :] + jnp.dot(
            h1.astype(jnp.bfloat16), whh1_ref[...],
            preferred_element_type=jnp.float32)
        h1, c1 = cell(g1, c1)
        z2 = jnp.concatenate([h1, h2], axis=1).astype(jnp.bfloat16)
        g2 = jnp.dot(z2, w2cat_ref[...],
                     preferred_element_type=jnp.float32) + b2_ref[...]
        h2, c2 = cell(g2, c2)
        hs.append(h2)
    h1_ref[...], c1_ref[...] = h1, c1
    h2_ref[...], c2_ref[...] = h2, c2
    hcat = jnp.concatenate(hs, axis=1)
    col = pl.multiple_of(s * K * H, K * H)
    rnn_ref[:, pl.ds(col, K * H)] = hcat
    acc_ref[...] += jnp.dot(hcat.astype(jnp.bfloat16),
                            w1r_ref[...].reshape(K * H, w1r_ref.shape[2]),
                            preferred_element_type=jnp.float32)
    acc = acc_ref[...]

    @pl.when(s == S - 1)
    def _():
        z1 = _lrelu(acc + c0b_ref[...])
        z2 = _lrelu(jnp.dot(z1.astype(jnp.bfloat16), c1w_ref[...],
                            preferred_element_type=jnp.float32) + c1b_ref[...])
        z3 = _lrelu(jnp.dot(z2.astype(jnp.bfloat16), c2w_ref[...],
                            preferred_element_type=jnp.float32) + c2b_ref[...])
        logits_ref[...] = jnp.dot(z3.astype(jnp.bfloat16), c3w_ref[...],
                                  preferred_element_type=jnp.float32) + c3b_ref[...]


def _lstm_classifier(x_tm, wih1, b1, whh1, w2cat, b2, w1r, cls, unroll):
    TB, I = x_tm.shape
    H = whh1.shape[0]
    N1 = w1r.shape[2]
    T = w1r.shape[0]
    BP = TB // T
    K = unroll
    c0b, c1w, c1b, c2w, c2b, c3w, c3b = cls
    n_out = c3w.shape[1]

    whole = lambda a: pl.BlockSpec(a.shape, lambda t: (0, 0))
    kern = functools.partial(_lstm_cls_kernel, T=T, H=H, BP=BP, K=K)
    return pl.pallas_call(
        kern,
        out_shape=[jax.ShapeDtypeStruct((BP, n_out), jnp.float32),
                   jax.ShapeDtypeStruct((BP, T * H), jnp.float32)],
        grid=(T // K,),
        in_specs=[
            whole(x_tm), whole(wih1), whole(b1), whole(whh1), whole(w2cat),
            whole(b2),
            pl.BlockSpec((K, H, N1), lambda s: (s, 0, 0)),
            whole(c0b), whole(c1w), whole(c1b), whole(c2w), whole(c2b),
            whole(c3w), whole(c3b),
        ],
        out_specs=[pl.BlockSpec((BP, n_out), lambda t: (0, 0)),
                   pl.BlockSpec((BP, T * H), lambda t: (0, 0))],
        scratch_shapes=[pltpu.VMEM((TB, 4 * H), jnp.float32),
                        pltpu.VMEM((BP, H), jnp.float32),
                        pltpu.VMEM((BP, H), jnp.float32),
                        pltpu.VMEM((BP, H), jnp.float32),
                        pltpu.VMEM((BP, H), jnp.float32),
                        pltpu.VMEM((BP, N1), jnp.float32)],
        compiler_params=pltpu.CompilerParams(
            dimension_semantics=("arbitrary",),
            vmem_limit_bytes=VMEM_LIMIT),
    )(x_tm, wih1, b1, whh1, w2cat, b2, w1r,
      c0b, c1w, c1b, c2w, c2b, c3w, c3b)


def kernel(x, fe0_w0, fe0_w1, fe0_w2, fe0_scale, fe0_shift, fe1_w0, fe1_w1, fe1_w2, fe1_scale, fe1_shift, fe2_w0, fe2_w1, fe2_w2, fe2_scale, fe2_shift, fe3_w0, fe3_w1, fe3_w2, fe3_scale, fe3_shift, rc0_w0, rc0_w1, rc0_w2, rc0_scale, rc0_shift, rc1_w0, rc1_w1, rc1_w2, rc1_scale, rc1_shift, rc2_w0, rc2_w1, rc2_w2, rc2_scale, rc2_shift, rc3_w0, rc3_w1, rc3_w2, rc3_scale, rc3_shift, cm0_w0, cm0_w1, cm0_w2, cm0_scale, cm0_shift, cm1_w0, cm1_w1, cm1_w2, cm1_scale, cm1_shift, cm2_w0, cm2_w1, cm2_w2, cm2_scale, cm2_shift, cm3_w0, cm3_w1, cm3_w2, cm3_scale, cm3_shift, lstm_wih1, lstm_b1, lstm_w1cat, lstm_whh2, lstm_b2, cls0_w_t, cls0_b, cls1_w_t, cls1_b, cls2_w_t, cls2_b, cls3_w_t, cls3_b):
    B, Cin, L = x.shape
    h = jnp.transpose(x, (0, 2, 1)).reshape(B * L, Cin)

    layers = [
        (fe0_w0, fe0_w1, fe0_w2, fe0_scale, fe0_shift),
        (fe1_w0, fe1_w1, fe1_w2, fe1_scale, fe1_shift),
        (fe2_w0, fe2_w1, fe2_w2, fe2_scale, fe2_shift),
        (fe3_w0, fe3_w1, fe3_w2, fe3_scale, fe3_shift),
        (rc0_w0, rc0_w1, rc0_w2, rc0_scale, rc0_shift),
        (rc1_w0, rc1_w1, rc1_w2, rc1_scale, rc1_shift),
        (rc2_w0, rc2_w1, rc2_w2, rc2_scale, rc2_shift),
        (rc3_w0, rc3_w1, rc3_w2, rc3_scale, rc3_shift),
        (cm0_w0, cm0_w1, cm0_w2, cm0_scale, cm0_shift),
        (cm1_w0, cm1_w1, cm1_w2, cm1_scale, cm1_shift),
        (cm2_w0, cm2_w1, cm2_w2, cm2_scale, cm2_shift),
        (cm3_w0, cm3_w1, cm3_w2, cm3_scale, cm3_shift),
    ]
    rc_out, cm_out = _trunk(h, layers, L)
    cons_input = jnp.transpose(rc_out.reshape(B, L, 2), (0, 2, 1))

    T = cm_out.shape[1]
    I = L
    H = lstm_whh2.shape[0]
    x3 = jnp.transpose(cm_out.reshape(B, L, T), (2, 0, 1))
    BP = max(8, ((B + 7) // 8) * 8)
    if BP > B:
        x3 = jnp.concatenate(
            [x3, jnp.zeros((T, BP - B, I), x3.dtype)], axis=1)

    N1 = cls0_w_t.shape[1]
    w1r = cls0_w_t.reshape(T, H, N1)
    whh1 = lstm_w1cat[:, :4 * H]
    w2cat = jnp.concatenate([lstm_w1cat[:, 4 * H:], lstm_whh2], axis=0)

    logits, rnn_p = _lstm_classifier(
        x3.reshape(T * BP, I), lstm_wih1, lstm_b1, whh1, w2cat,
        lstm_b2, w1r,
        (cls0_b, cls1_w_t, cls1_b, cls2_w_t, cls2_b, cls3_w_t, cls3_b),
        unroll=25)
    cons_input = cons_input
    rnn_feature = rnn_p[:B]
    logits = logits[:B]
    return logits, rnn_feature, cons_input


# speedup vs baseline: 1.0185x; 1.0185x over previous
import functools

import jax
import jax.numpy as jnp
from jax.experimental import pallas as pl
from jax.experimental.pallas import tpu as pltpu

LRELU_SLOPE = 0.01
VMEM_LIMIT = 48 * 1024 * 1024


def _lrelu(y):
    return jnp.where(y > 0, y, LRELU_SLOPE * y)


def _conv3(x, w0, w1, w2, scale, shift, is_start, is_end, act):
    m = x.shape[0]
    xb = x if x.dtype == jnp.bfloat16 else x.astype(jnp.bfloat16)
    y1 = jnp.dot(xb, w1, preferred_element_type=jnp.float32)
    y0 = jnp.dot(xb, w0, preferred_element_type=jnp.float32)
    y2 = jnp.dot(xb, w2, preferred_element_type=jnp.float32)
    acc = (y1
           + jnp.where(is_start, 0.0, pltpu.roll(y0, 1, axis=0))
           + jnp.where(is_end, 0.0, pltpu.roll(y2, m - 1, axis=0)))
    y = acc * scale + shift
    return _lrelu(y) if act else y


def _trunk_kernel(x_ref, *refs, seq_len):
    wr = refs[:60]
    rc_o, cm_o = refs[60], refs[61]
    s50, s256, s512, s_sh, s128 = refs[62:]

    m = x_ref.shape[0]
    row = jax.lax.broadcasted_iota(jnp.int32, (m, 1), 0)
    is_start = (row % seq_len) == 0
    is_end = (row % seq_len) == (seq_len - 1)

    def layer(src_ref, dst_ref, li, act):
        w0, w1, w2, scale, shift = (r[...] for r in wr[5 * li:5 * li + 5])
        y = _conv3(src_ref[...], w0, w1, w2, scale, shift,
                   is_start, is_end, act)
        dst_ref[...] = y.astype(dst_ref.dtype)

    layer(x_ref, s50, 0, True)
    layer(s50, s256, 1, True)
    layer(s256, s512, 2, True)
    layer(s512, s_sh, 3, True)
    layer(s_sh, s512, 4, True)
    layer(s512, s256, 5, True)
    layer(s256, s50, 6, True)
    layer(s50, rc_o, 7, False)
    layer(s_sh, s512, 8, True)
    layer(s512, s256, 9, True)
    layer(s256, s128, 10, True)
    layer(s128, cm_o, 11, True)


def _trunk(x_2d, layer_params, seq_len):
    M, cin0 = x_2d.shape
    n_blocks = 2 if M % (2 * seq_len) == 0 else 1
    bm = M // n_blocks

    in_specs = [pl.BlockSpec((bm, cin0), lambda i: (i, 0))]
    args = [x_2d]
    for p in layer_params:
        for a in p:
            in_specs.append(pl.BlockSpec(a.shape, lambda i: (0, 0)))
            args.append(a)

    kern = functools.partial(_trunk_kernel, seq_len=seq_len)
    return pl.pallas_call(
        kern,
        out_shape=[jax.ShapeDtypeStruct((M, 2), jnp.float32),
                   jax.ShapeDtypeStruct((M, 50), jnp.bfloat16)],
        grid=(n_blocks,),
        in_specs=in_specs,
        out_specs=[pl.BlockSpec((bm, 2), lambda i: (i, 0)),
                   pl.BlockSpec((bm, 50), lambda i: (i, 0))],
        scratch_shapes=[pltpu.VMEM((bm, 50), jnp.bfloat16),
                        pltpu.VMEM((bm, 256), jnp.bfloat16),
                        pltpu.VMEM((bm, 512), jnp.bfloat16),
                        pltpu.VMEM((bm, 1024), jnp.bfloat16),
                        pltpu.VMEM((bm, 128), jnp.bfloat16)],
        compiler_params=pltpu.CompilerParams(
            dimension_semantics=("arbitrary",),
            vmem_limit_bytes=VMEM_LIMIT),
    )(*args)


def _lstm_cls_kernel(x_ref, wih1_ref, b1_ref, whh1_ref, w2cat_ref, b2_ref,
                     w1r_ref,
                     c0b_ref, c1w_ref, c1b_ref, c2w_ref, c2b_ref,
                     c3w_ref, c3b_ref,
                     logits_ref, rnn_ref,
                     gx_ref, h1_ref, c1_ref, h2_ref, c2_ref, acc_ref,
                     *, T, H, BP, K):
    s = pl.program_id(0)
    S = T // K

    @pl.when(s == 0)
    def _():
        h1_ref[...] = jnp.zeros_like(h1_ref)
        c1_ref[...] = jnp.zeros_like(c1_ref)
        h2_ref[...] = jnp.zeros_like(h2_ref)
        c2_ref[...] = jnp.zeros_like(c2_ref)
        acc_ref[...] = jnp.zeros_like(acc_ref)
        gx_ref[...] = jnp.dot(x_ref[...].astype(jnp.bfloat16), wih1_ref[...],
                              preferred_element_type=jnp.float32) + b1_ref[...]

    def cell(g, c):
        i_g = jax.nn.sigmoid(g[:, 0:H])
        f_g = jax.nn.sigmoid(g[:, H:2 * H])
        g_g = jnp.tanh(g[:, 2 * H:3 * H])
        o_g = jax.nn.sigmoid(g[:, 3 * H:4 * H])
        c_new = f_g * c + i_g * g_g
        return o_g * jnp.tanh(c_new), c_new

    h1, c1 = h1_ref[...], c1_ref[...]
    h2, c2 = h2_ref[...], c2_ref[...]
    hs = []
    for j in range(K):
        row = pl.multiple_of((s * K + j) * BP, BP)
        g1 = gx_ref[pl.ds(row, BP), :] + jnp.dot(
            h1.astype(jnp.bfloat16), whh1_ref[...],
            preferred_element_type=jnp.float32)
        h1, c1 = cell(g1, c1)
        z2 = jnp.concatenate([h1, h2], axis=1).astype(jnp.bfloat16)
        g2 = jnp.dot(z2, w2cat_ref[...],
                     preferred_element_type=jnp.float32) + b2_ref[...]
        h2, c2 = cell(g2, c2)
        hs.append(h2)
    h1_ref[...], c1_ref[...] = h1, c1
    h2_ref[...], c2_ref[...] = h2, c2
    hcat = jnp.concatenate(hs, axis=1)
    col = pl.multiple_of(s * K * H, K * H)
    rnn_ref[:, pl.ds(col, K * H)] = hcat
    acc_ref[...] += jnp.dot(hcat.astype(jnp.bfloat16),
                            w1r_ref[...].reshape(K * H, w1r_ref.shape[2]),
                            preferred_element_type=jnp.float32)
    acc = acc_ref[...]

    @pl.when(s == S - 1)
    def _():
        z1 = _lrelu(acc + c0b_ref[...])
        z2 = _lrelu(jnp.dot(z1.astype(jnp.bfloat16), c1w_ref[...],
                            preferred_element_type=jnp.float32) + c1b_ref[...])
        z3 = _lrelu(jnp.dot(z2.astype(jnp.bfloat16), c2w_ref[...],
                            preferred_element_type=jnp.float32) + c2b_ref[...])
        logits_ref[...] = jnp.dot(z3.astype(jnp.bfloat16), c3w_ref[...],
                                  preferred_element_type=jnp.float32) + c3b_ref[...]


def _lstm_classifier(x_tm, wih1, b1, whh1, w2cat, b2, w1r, cls, unroll):
    TB, I = x_tm.shape
    H = whh1.shape[0]
    N1 = w1r.shape[2]
    T = w1r.shape[0]
    BP = TB // T
    K = unroll
    c0b, c1w, c1b, c2w, c2b, c3w, c3b = cls
    n_out = c3w.shape[1]

    whole = lambda a: pl.BlockSpec(a.shape, lambda t: (0, 0))
    kern = functools.partial(_lstm_cls_kernel, T=T, H=H, BP=BP, K=K)
    return pl.pallas_call(
        kern,
        out_shape=[jax.ShapeDtypeStruct((BP, n_out), jnp.float32),
                   jax.ShapeDtypeStruct((BP, T * H), jnp.float32)],
        grid=(T // K,),
        in_specs=[
            whole(x_tm), whole(wih1), whole(b1), whole(whh1), whole(w2cat),
            whole(b2),
            pl.BlockSpec((K, H, N1), lambda s: (s, 0, 0)),
            whole(c0b), whole(c1w), whole(c1b), whole(c2w), whole(c2b),
            whole(c3w), whole(c3b),
        ],
        out_specs=[pl.BlockSpec((BP, n_out), lambda t: (0, 0)),
                   pl.BlockSpec((BP, T * H), lambda t: (0, 0))],
        scratch_shapes=[pltpu.VMEM((TB, 4 * H), jnp.float32),
                        pltpu.VMEM((BP, H), jnp.float32),
                        pltpu.VMEM((BP, H), jnp.float32),
                        pltpu.VMEM((BP, H), jnp.float32),
                        pltpu.VMEM((BP, H), jnp.float32),
                        pltpu.VMEM((BP, N1), jnp.float32)],
        compiler_params=pltpu.CompilerParams(
            dimension_semantics=("arbitrary",),
            vmem_limit_bytes=VMEM_LIMIT),
    )(x_tm, wih1, b1, whh1, w2cat, b2, w1r,
      c0b, c1w, c1b, c2w, c2b, c3w, c3b)


def kernel(x, fe0_w0, fe0_w1, fe0_w2, fe0_scale, fe0_shift, fe1_w0, fe1_w1, fe1_w2, fe1_scale, fe1_shift, fe2_w0, fe2_w1, fe2_w2, fe2_scale, fe2_shift, fe3_w0, fe3_w1, fe3_w2, fe3_scale, fe3_shift, rc0_w0, rc0_w1, rc0_w2, rc0_scale, rc0_shift, rc1_w0, rc1_w1, rc1_w2, rc1_scale, rc1_shift, rc2_w0, rc2_w1, rc2_w2, rc2_scale, rc2_shift, rc3_w0, rc3_w1, rc3_w2, rc3_scale, rc3_shift, cm0_w0, cm0_w1, cm0_w2, cm0_scale, cm0_shift, cm1_w0, cm1_w1, cm1_w2, cm1_scale, cm1_shift, cm2_w0, cm2_w1, cm2_w2, cm2_scale, cm2_shift, cm3_w0, cm3_w1, cm3_w2, cm3_scale, cm3_shift, lstm_wih1, lstm_b1, lstm_w1cat, lstm_whh2, lstm_b2, cls0_w_t, cls0_b, cls1_w_t, cls1_b, cls2_w_t, cls2_b, cls3_w_t, cls3_b):
    B, Cin, L = x.shape
    h = jnp.transpose(x, (0, 2, 1)).reshape(B * L, Cin)

    layers = [
        (fe0_w0, fe0_w1, fe0_w2, fe0_scale, fe0_shift),
        (fe1_w0, fe1_w1, fe1_w2, fe1_scale, fe1_shift),
        (fe2_w0, fe2_w1, fe2_w2, fe2_scale, fe2_shift),
        (fe3_w0, fe3_w1, fe3_w2, fe3_scale, fe3_shift),
        (rc0_w0, rc0_w1, rc0_w2, rc0_scale, rc0_shift),
        (rc1_w0, rc1_w1, rc1_w2, rc1_scale, rc1_shift),
        (rc2_w0, rc2_w1, rc2_w2, rc2_scale, rc2_shift),
        (rc3_w0, rc3_w1, rc3_w2, rc3_scale, rc3_shift),
        (cm0_w0, cm0_w1, cm0_w2, cm0_scale, cm0_shift),
        (cm1_w0, cm1_w1, cm1_w2, cm1_scale, cm1_shift),
        (cm2_w0, cm2_w1, cm2_w2, cm2_scale, cm2_shift),
        (cm3_w0, cm3_w1, cm3_w2, cm3_scale, cm3_shift),
    ]
    rc_out, cm_out = _trunk(h, layers, L)
    cons_input = jnp.transpose(rc_out.reshape(B, L, 2), (0, 2, 1))

    T = cm_out.shape[1]
    I = L
    H = lstm_whh2.shape[0]
    x3 = jnp.transpose(cm_out.reshape(B, L, T), (2, 0, 1))
    BP = max(8, ((B + 7) // 8) * 8)
    if BP > B:
        x3 = jnp.concatenate(
            [x3, jnp.zeros((T, BP - B, I), x3.dtype)], axis=1)

    N1 = cls0_w_t.shape[1]
    w1r = cls0_w_t.reshape(T, H, N1)
    whh1 = lstm_w1cat[:, :4 * H]
    w2cat = jnp.concatenate([lstm_w1cat[:, 4 * H:], lstm_whh2], axis=0)

    logits, rnn_p = _lstm_classifier(
        x3.reshape(T * BP, I), lstm_wih1, lstm_b1, whh1, w2cat,
        lstm_b2, w1r,
        (cls0_b, cls1_w_t, cls1_b, cls2_w_t, cls2_b, cls3_w_t, cls3_b),
        unroll=10)
    cons_input = cons_input
    rnn_feature = rnn_p[:B]
    logits = logits[:B]
    return logits, rnn_feature, cons_input


# speedup vs baseline: 1.0209x; 1.0025x over previous
import functools

import jax
import jax.numpy as jnp
from jax.experimental import pallas as pl
from jax.experimental.pallas import tpu as pltpu

LRELU_SLOPE = 0.01
VMEM_LIMIT = 48 * 1024 * 1024


def _lrelu(y):
    return jnp.where(y > 0, y, LRELU_SLOPE * y)


def _conv3(x, w0, w1, w2, scale, shift, is_start, is_end, act):
    m = x.shape[0]
    xb = x if x.dtype == jnp.bfloat16 else x.astype(jnp.bfloat16)
    y1 = jnp.dot(xb, w1, preferred_element_type=jnp.float32)
    y0 = jnp.dot(xb, w0, preferred_element_type=jnp.float32)
    y2 = jnp.dot(xb, w2, preferred_element_type=jnp.float32)
    acc = (y1
           + jnp.where(is_start, 0.0, pltpu.roll(y0, 1, axis=0))
           + jnp.where(is_end, 0.0, pltpu.roll(y2, m - 1, axis=0)))
    y = acc * scale + shift
    return _lrelu(y) if act else y


def _trunk_kernel(x_ref, ss_ref, *refs, seq_len):
    wr = refs[:36]
    rc_o, cm_o = refs[36], refs[37]
    s50, s256, s512, s_sh, s128 = refs[38:]

    m = x_ref.shape[0]
    row = jax.lax.broadcasted_iota(jnp.int32, (m, 1), 0)
    is_start = (row % seq_len) == 0
    is_end = (row % seq_len) == (seq_len - 1)

    def layer(src_ref, dst_ref, li, act):
        w0, w1, w2 = (r[...] for r in wr[3 * li:3 * li + 3])
        cout = w0.shape[1]
        scale = ss_ref[2 * li:2 * li + 1, :cout]
        shift = ss_ref[2 * li + 1:2 * li + 2, :cout]
        y = _conv3(src_ref[...], w0, w1, w2, scale, shift,
                   is_start, is_end, act)
        dst_ref[...] = y.astype(dst_ref.dtype)

    layer(x_ref, s50, 0, True)
    layer(s50, s256, 1, True)
    layer(s256, s512, 2, True)
    layer(s512, s_sh, 3, True)
    layer(s_sh, s512, 4, True)
    layer(s512, s256, 5, True)
    layer(s256, s50, 6, True)
    layer(s50, rc_o, 7, False)
    layer(s_sh, s512, 8, True)
    layer(s512, s256, 9, True)
    layer(s256, s128, 10, True)
    layer(s128, cm_o, 11, True)


def _trunk(x_2d, layer_params, seq_len):
    M, cin0 = x_2d.shape
    n_blocks = 2 if M % (2 * seq_len) == 0 else 1
    bm = M // n_blocks

    nmax = max(p[1].shape[1] for p in layer_params)
    ss = jnp.concatenate(
        [jnp.pad(a, ((0, 0), (0, nmax - a.shape[1])))
         for p in layer_params for a in p[3:]], axis=0)

    in_specs = [pl.BlockSpec((bm, cin0), lambda i: (i, 0)),
                pl.BlockSpec(ss.shape, lambda i: (0, 0))]
    args = [x_2d, ss]
    for p in layer_params:
        for a in p[:3]:
            in_specs.append(pl.BlockSpec(a.shape, lambda i: (0, 0)))
            args.append(a)

    kern = functools.partial(_trunk_kernel, seq_len=seq_len)
    return pl.pallas_call(
        kern,
        out_shape=[jax.ShapeDtypeStruct((M, 2), jnp.float32),
                   jax.ShapeDtypeStruct((M, 50), jnp.bfloat16)],
        grid=(n_blocks,),
        in_specs=in_specs,
        out_specs=[pl.BlockSpec((bm, 2), lambda i: (i, 0)),
                   pl.BlockSpec((bm, 50), lambda i: (i, 0))],
        scratch_shapes=[pltpu.VMEM((bm, 50), jnp.bfloat16),
                        pltpu.VMEM((bm, 256), jnp.bfloat16),
                        pltpu.VMEM((bm, 512), jnp.bfloat16),
                        pltpu.VMEM((bm, 1024), jnp.bfloat16),
                        pltpu.VMEM((bm, 128), jnp.bfloat16)],
        compiler_params=pltpu.CompilerParams(
            dimension_semantics=("arbitrary",),
            vmem_limit_bytes=VMEM_LIMIT),
    )(*args)


def _lstm_cls_kernel(x_ref, wih1_ref, b1_ref, whh1_ref, w2cat_ref, b2_ref,
                     w1r_ref,
                     c0b_ref, c1w_ref, c1b_ref, c2w_ref, c2b_ref,
                     c3w_ref, c3b_ref,
                     logits_ref, rnn_ref,
                     gx_ref, h1_ref, c1_ref, h2_ref, c2_ref, acc_ref,
                     *, T, H, BP, K):
    s = pl.program_id(0)
    S = T // K

    @pl.when(s == 0)
    def _():
        h1_ref[...] = jnp.zeros_like(h1_ref)
        c1_ref[...] = jnp.zeros_like(c1_ref)
        h2_ref[...] = jnp.zeros_like(h2_ref)
        c2_ref[...] = jnp.zeros_like(c2_ref)
        acc_ref[...] = jnp.zeros_like(acc_ref)
        gx_ref[...] = jnp.dot(x_ref[...].astype(jnp.bfloat16), wih1_ref[...],
                              preferred_element_type=jnp.float32) + b1_ref[...]

    def cell(g, c):
        i_g = jax.nn.sigmoid(g[:, 0:H])
        f_g = jax.nn.sigmoid(g[:, H:2 * H])
        g_g = jnp.tanh(g[:, 2 * H:3 * H])
        o_g = jax.nn.sigmoid(g[:, 3 * H:4 * H])
        c_new = f_g * c + i_g * g_g
        return o_g * jnp.tanh(c_new), c_new

    h1, c1 = h1_ref[...], c1_ref[...]
    h2, c2 = h2_ref[...], c2_ref[...]
    hs = []
    for j in range(K):
        row = pl.multiple_of((s * K + j) * BP, BP)
        g1 = gx_ref[pl.ds(row, BP), :] + jnp.dot(
            h1.astype(jnp.bfloat16), whh1_ref[...],
            preferred_element_type=jnp.float32)
        h1, c1 = cell(g1, c1)
        z2 = jnp.concatenate([h1, h2], axis=1).astype(jnp.bfloat16)
        g2 = jnp.dot(z2, w2cat_ref[...],
                     preferred_element_type=jnp.float32) + b2_ref[...]
        h2, c2 = cell(g2, c2)
        hs.append(h2)
    h1_ref[...], c1_ref[...] = h1, c1
    h2_ref[...], c2_ref[...] = h2, c2
    hcat = jnp.concatenate(hs, axis=1)
    col = pl.multiple_of(s * K * H, K * H)
    rnn_ref[:, pl.ds(col, K * H)] = hcat
    acc_ref[...] += jnp.dot(hcat.astype(jnp.bfloat16),
                            w1r_ref[...].reshape(K * H, w1r_ref.shape[2]),
                            preferred_element_type=jnp.float32)
    acc = acc_ref[...]

    @pl.when(s == S - 1)
    def _():
        z1 = _lrelu(acc + c0b_ref[...])
        z2 = _lrelu(jnp.dot(z1.astype(jnp.bfloat16), c1w_ref[...],
                            preferred_element_type=jnp.float32) + c1b_ref[...])
        z3 = _lrelu(jnp.dot(z2.astype(jnp.bfloat16), c2w_ref[...],
                            preferred_element_type=jnp.float32) + c2b_ref[...])
        logits_ref[...] = jnp.dot(z3.astype(jnp.bfloat16), c3w_ref[...],
                                  preferred_element_type=jnp.float32) + c3b_ref[...]


def _lstm_classifier(x_tm, wih1, b1, whh1, w2cat, b2, w1r, cls, unroll):
    TB, I = x_tm.shape
    H = whh1.shape[0]
    N1 = w1r.shape[2]
    T = w1r.shape[0]
    BP = TB // T
    K = unroll
    c0b, c1w, c1b, c2w, c2b, c3w, c3b = cls
    n_out = c3w.shape[1]

    whole = lambda a: pl.BlockSpec(a.shape, lambda t: (0, 0))
    kern = functools.partial(_lstm_cls_kernel, T=T, H=H, BP=BP, K=K)
    return pl.pallas_call(
        kern,
        out_shape=[jax.ShapeDtypeStruct((BP, n_out), jnp.float32),
                   jax.ShapeDtypeStruct((BP, T * H), jnp.float32)],
        grid=(T // K,),
        in_specs=[
            whole(x_tm), whole(wih1), whole(b1), whole(whh1), whole(w2cat),
            whole(b2),
            pl.BlockSpec((K, H, N1), lambda s: (s, 0, 0)),
            whole(c0b), whole(c1w), whole(c1b), whole(c2w), whole(c2b),
            whole(c3w), whole(c3b),
        ],
        out_specs=[pl.BlockSpec((BP, n_out), lambda t: (0, 0)),
                   pl.BlockSpec((BP, T * H), lambda t: (0, 0))],
        scratch_shapes=[pltpu.VMEM((TB, 4 * H), jnp.float32),
                        pltpu.VMEM((BP, H), jnp.float32),
                        pltpu.VMEM((BP, H), jnp.float32),
                        pltpu.VMEM((BP, H), jnp.float32),
                        pltpu.VMEM((BP, H), jnp.float32),
                        pltpu.VMEM((BP, N1), jnp.float32)],
        compiler_params=pltpu.CompilerParams(
            dimension_semantics=("arbitrary",),
            vmem_limit_bytes=VMEM_LIMIT),
    )(x_tm, wih1, b1, whh1, w2cat, b2, w1r,
      c0b, c1w, c1b, c2w, c2b, c3w, c3b)


def kernel(x, fe0_w0, fe0_w1, fe0_w2, fe0_scale, fe0_shift, fe1_w0, fe1_w1, fe1_w2, fe1_scale, fe1_shift, fe2_w0, fe2_w1, fe2_w2, fe2_scale, fe2_shift, fe3_w0, fe3_w1, fe3_w2, fe3_scale, fe3_shift, rc0_w0, rc0_w1, rc0_w2, rc0_scale, rc0_shift, rc1_w0, rc1_w1, rc1_w2, rc1_scale, rc1_shift, rc2_w0, rc2_w1, rc2_w2, rc2_scale, rc2_shift, rc3_w0, rc3_w1, rc3_w2, rc3_scale, rc3_shift, cm0_w0, cm0_w1, cm0_w2, cm0_scale, cm0_shift, cm1_w0, cm1_w1, cm1_w2, cm1_scale, cm1_shift, cm2_w0, cm2_w1, cm2_w2, cm2_scale, cm2_shift, cm3_w0, cm3_w1, cm3_w2, cm3_scale, cm3_shift, lstm_wih1, lstm_b1, lstm_w1cat, lstm_whh2, lstm_b2, cls0_w_t, cls0_b, cls1_w_t, cls1_b, cls2_w_t, cls2_b, cls3_w_t, cls3_b):
    B, Cin, L = x.shape
    h = jnp.transpose(x, (0, 2, 1)).reshape(B * L, Cin)

    layers = [
        (fe0_w0, fe0_w1, fe0_w2, fe0_scale, fe0_shift),
        (fe1_w0, fe1_w1, fe1_w2, fe1_scale, fe1_shift),
        (fe2_w0, fe2_w1, fe2_w2, fe2_scale, fe2_shift),
        (fe3_w0, fe3_w1, fe3_w2, fe3_scale, fe3_shift),
        (rc0_w0, rc0_w1, rc0_w2, rc0_scale, rc0_shift),
        (rc1_w0, rc1_w1, rc1_w2, rc1_scale, rc1_shift),
        (rc2_w0, rc2_w1, rc2_w2, rc2_scale, rc2_shift),
        (rc3_w0, rc3_w1, rc3_w2, rc3_scale, rc3_shift),
        (cm0_w0, cm0_w1, cm0_w2, cm0_scale, cm0_shift),
        (cm1_w0, cm1_w1, cm1_w2, cm1_scale, cm1_shift),
        (cm2_w0, cm2_w1, cm2_w2, cm2_scale, cm2_shift),
        (cm3_w0, cm3_w1, cm3_w2, cm3_scale, cm3_shift),
    ]
    rc_out, cm_out = _trunk(h, layers, L)
    cons_input = jnp.transpose(rc_out.reshape(B, L, 2), (0, 2, 1))

    T = cm_out.shape[1]
    I = L
    H = lstm_whh2.shape[0]
    x3 = jnp.transpose(cm_out.reshape(B, L, T), (2, 0, 1))
    BP = max(8, ((B + 7) // 8) * 8)
    if BP > B:
        x3 = jnp.concatenate(
            [x3, jnp.zeros((T, BP - B, I), x3.dtype)], axis=1)

    N1 = cls0_w_t.shape[1]
    w1r = cls0_w_t.reshape(T, H, N1)
    whh1 = lstm_w1cat[:, :4 * H]
    w2cat = jnp.concatenate([lstm_w1cat[:, 4 * H:], lstm_whh2], axis=0)

    logits, rnn_p = _lstm_classifier(
        x3.reshape(T * BP, I), lstm_wih1, lstm_b1, whh1, w2cat,
        lstm_b2, w1r,
        (cls0_b, cls1_w_t, cls1_b, cls2_w_t, cls2_b, cls3_w_t, cls3_b),
        unroll=10)
    cons_input = cons_input
    rnn_feature = rnn_p[:B]
    logits = logits[:B]
    return logits, rnn_feature, cons_input
